# Initial kernel scaffold; baseline (speedup 1.0000x reference)
import jax
import jax.numpy as jnp
from jax.experimental import pallas as pl


def kernel(x, plane_xy, plane_xz, plane_yz):
    raise NotImplementedError("write your pallas kernel here")


# speedup vs baseline: 1.4644x; 1.4644x over previous
import functools

import jax
import jax.numpy as jnp
from jax import lax
from jax.experimental import pallas as pl
from jax.experimental.pallas import tpu as pltpu
from jax.experimental.pallas import tpu_sc as plsc

R = 512
F = 16
L = 16
NC = 2
NS = 16
NW = NC * NS
C = 128
BN = 4096


def _minmax_body(x_ref, mn_ref, mx_ref):
    i = pl.program_id(0)
    xb = x_ref[...]
    mn = jnp.broadcast_to(jnp.min(xb, axis=1, keepdims=True), (3, 128))
    mx = jnp.broadcast_to(jnp.max(xb, axis=1, keepdims=True), (3, 128))

    @pl.when(i == 0)
    def _():
        mn_ref[...] = mn
        mx_ref[...] = mx

    @pl.when(i > 0)
    def _():
        mn_ref[...] = jnp.minimum(mn_ref[...], mn)
        mx_ref[...] = jnp.maximum(mx_ref[...], mx)


def _idxw_body(x_ref, mn_ref, mx_ref, idx_ref, wx_ref, wy_ref):
    xb = x_ref[...]
    mn3 = mn_ref[:, 0:1]
    mx3 = mx_ref[:, 0:1]
    scale = (R - 1.0) / (mx3 - mn3 + 1e-8)
    ic = (xb - mn3) * scale
    i0f = jnp.clip(jnp.floor(ic), 0.0, R - 2.0)
    w = ic - i0f
    ii = i0f.astype(jnp.int32)
    idx_ref[0:1, :] = ii[1:2, :] * R + ii[0:1, :]
    idx_ref[1:2, :] = R * R + ii[2:3, :] * R + ii[0:1, :]
    idx_ref[2:3, :] = 2 * R * R + ii[2:3, :] * R + ii[1:2, :]
    wx_ref[0:1, :] = w[0:1, :]
    wx_ref[1:2, :] = w[0:1, :]
    wx_ref[2:3, :] = w[1:2, :]
    wy_ref[0:1, :] = w[1:2, :]
    wy_ref[1:2, :] = w[2:3, :]
    wy_ref[2:3, :] = w[2:3, :]


def _make_sc_kernel(n_pad):
    nw_pts = n_pad // NW
    n_chunks = nw_pts // C
    mesh = plsc.VectorSubcoreMesh(core_axis_name="c", subcore_axis_name="s")

    @functools.partial(
        pl.kernel,
        mesh=mesh,
        compiler_params=pltpu.CompilerParams(use_tc_tiling_on_sc=False),
        out_type=jax.ShapeDtypeStruct((n_pad, 3 * F), jnp.float32),
        scratch_types=[
            pltpu.VMEM((C,), jnp.int32),
            pltpu.VMEM((C,), jnp.int32),
            pltpu.VMEM((C,), jnp.int32),
            pltpu.VMEM((C,), jnp.int32),
            pltpu.VMEM((C, F), jnp.float32),
            pltpu.VMEM((C, F), jnp.float32),
            pltpu.VMEM((C, F), jnp.float32),
            pltpu.VMEM((C, F), jnp.float32),
            pltpu.VMEM((C,), jnp.float32),
            pltpu.VMEM((C,), jnp.float32),
            pltpu.VMEM((C, 3 * F), jnp.float32),
            pltpu.SemaphoreType.DMA,
        ],
    )
    def sc_kernel(table, idxh, wxh, wyh, outh,
                  i0, i1, i2, i3, c0, c1, c2, c3, wxv, wyv, oc, sem):
        wid = lax.axis_index("s") * NC + lax.axis_index("c")

        def chunk_body(g, carry):
            base = wid * nw_pts + g * C
            for p in range(3):
                pltpu.sync_copy(idxh.at[pl.ds(p * n_pad + base, C)], i0)
                pltpu.sync_copy(wxh.at[pl.ds(p * n_pad + base, C)], wxv)
                pltpu.sync_copy(wyh.at[pl.ds(p * n_pad + base, C)], wyv)
                for q in range(C // L):
                    s = pl.ds(q * L, L)
                    v = i0[s]
                    i1[s] = v + 1
                    i2[s] = v + R
                    i3[s] = v + (R + 1)
                cps = [pltpu.async_copy(table.at[ib], cb, sem)
                       for ib, cb in ((i0, c0), (i1, c1), (i2, c2), (i3, c3))]
                for cp in cps:
                    cp.wait()

                def grp_body(t, cin):
                    s = pl.ds(t * L, L)
                    wx = wxv[s]
                    wy = wyv[s]
                    ax = 1.0 - wx
                    ay = 1.0 - wy
                    a0 = ax * ay
                    a1 = wx * ay
                    a2 = ax * wy
                    a3 = wx * wy
                    for u in range(L):
                        pt = t * L + u
                        acc = (c0[pt, :] * a0[u] + c1[pt, :] * a1[u]
                               + c2[pt, :] * a2[u] + c3[pt, :] * a3[u])
                        oc[pt, pl.ds(p * F, F)] = acc
                    return cin

                lax.fori_loop(0, C // L, grp_body, 0)
            pltpu.sync_copy(oc, outh.at[pl.ds(base, C), :])
            return carry

        lax.fori_loop(0, n_chunks, chunk_body, 0)

    return sc_kernel


def kernel(x, plane_xy, plane_xz, plane_yz):
    n = x.shape[0]
    n_pad = ((n + NW * C - 1) // (NW * C)) * (NW * C)
    n_pad = ((n_pad + BN - 1) // BN) * BN

    table = jnp.concatenate(
        [jnp.transpose(p[0], (1, 2, 0)).reshape(R * R, F)
         for p in (plane_xy, plane_xz, plane_yz)], axis=0)

    x_t = jnp.pad(jnp.transpose(x), ((0, 0), (0, n_pad - n)), mode="edge")

    grid = n_pad // BN
    mn, mx = pl.pallas_call(
        _minmax_body,
        grid=(grid,),
        in_specs=[pl.BlockSpec((3, BN), lambda i: (0, i))],
        out_specs=[pl.BlockSpec((3, 128), lambda i: (0, 0)),
                   pl.BlockSpec((3, 128), lambda i: (0, 0))],
        out_shape=[jax.ShapeDtypeStruct((3, 128), jnp.float32),
                   jax.ShapeDtypeStruct((3, 128), jnp.float32)],
    )(x_t)

    idx, wx, wy = pl.pallas_call(
        _idxw_body,
        grid=(grid,),
        in_specs=[pl.BlockSpec((3, BN), lambda i: (0, i)),
                  pl.BlockSpec((3, 128), lambda i: (0, 0)),
                  pl.BlockSpec((3, 128), lambda i: (0, 0))],
        out_specs=[pl.BlockSpec((3, BN), lambda i: (0, i)),
                   pl.BlockSpec((3, BN), lambda i: (0, i)),
                   pl.BlockSpec((3, BN), lambda i: (0, i))],
        out_shape=[jax.ShapeDtypeStruct((3, n_pad), jnp.int32),
                   jax.ShapeDtypeStruct((3, n_pad), jnp.float32),
                   jax.ShapeDtypeStruct((3, n_pad), jnp.float32)],
    )(x_t, mn, mx)

    out = _make_sc_kernel(n_pad)(table, idx.reshape(-1), wx.reshape(-1),
                                 wy.reshape(-1))
    return out[:n]


# speedup vs baseline: 2.2605x; 1.5437x over previous
import functools

import jax
import jax.numpy as jnp
from jax import lax
from jax.experimental import pallas as pl
from jax.experimental.pallas import tpu as pltpu
from jax.experimental.pallas import tpu_sc as plsc

R = 512
F = 16
L = 16
NC = 2
NS = 16
NW = NC * NS
CH = 256
HC = 128
BN = 4096
TBL = 3 * R * R


def _minmax_body(x_ref, mn_ref, mx_ref):
    i = pl.program_id(0)
    xb = x_ref[...]
    mn = jnp.broadcast_to(jnp.min(xb, axis=1, keepdims=True), (3, 128))
    mx = jnp.broadcast_to(jnp.max(xb, axis=1, keepdims=True), (3, 128))

    @pl.when(i == 0)
    def _():
        mn_ref[...] = mn
        mx_ref[...] = mx

    @pl.when(i > 0)
    def _():
        mn_ref[...] = jnp.minimum(mn_ref[...], mn)
        mx_ref[...] = jnp.maximum(mx_ref[...], mx)


def _idxw_body(x_ref, mn_ref, mx_ref, idx_ref, wx_ref, wy_ref):
    xb = x_ref[...]
    mn3 = mn_ref[:, 0:1]
    mx3 = mx_ref[:, 0:1]
    scale = (R - 1.0) / (mx3 - mn3 + 1e-8)
    ic = (xb - mn3) * scale
    i0f = jnp.clip(jnp.floor(ic), 0.0, R - 2.0)
    w = ic - i0f
    ii = i0f.astype(jnp.int32)
    idx_ref[0:1, :] = ii[1:2, :] * R + ii[0:1, :]
    idx_ref[1:2, :] = R * R + ii[2:3, :] * R + ii[0:1, :]
    idx_ref[2:3, :] = 2 * R * R + ii[2:3, :] * R + ii[1:2, :]
    wx_ref[0:1, :] = w[0:1, :]
    wx_ref[1:2, :] = w[0:1, :]
    wx_ref[2:3, :] = w[1:2, :]
    wy_ref[0:1, :] = w[1:2, :]
    wy_ref[1:2, :] = w[2:3, :]
    wy_ref[2:3, :] = w[2:3, :]


def _make_sc_kernel(n_pad):
    nw_pts = n_pad // NW
    n_chunks = nw_pts // CH
    n_steps = 3 * n_chunks
    loop_iters = (n_steps - 1) // 6
    mesh = plsc.VectorSubcoreMesh(core_axis_name="c", subcore_axis_name="s")

    @functools.partial(
        pl.kernel,
        mesh=mesh,
        compiler_params=pltpu.CompilerParams(use_tc_tiling_on_sc=False),
        out_type=jax.ShapeDtypeStruct((n_pad, 3 * F), jnp.float32),
        scratch_types=(
            [pltpu.VMEM((CH,), jnp.int32) for _ in range(3)]
            + [pltpu.VMEM((CH,), jnp.float32) for _ in range(6)]
            + [pltpu.VMEM((CH, F), jnp.float32) for _ in range(8)]
            + [pltpu.VMEM((CH, 3 * F), jnp.float32) for _ in range(2)]
            + [pltpu.SemaphoreType.DMA for _ in range(5)]
        ),
    )
    def sc_kernel(table, idxh, wxh, wyh, outh,
                  i0a, i0b, i0c, wxa, wxb, wxc, wya, wyb, wyc,
                  ca0, ca1, ca2, ca3, cb0, cb1, cb2, cb3,
                  oca, ocb, sin_a, sin_b, sin_c, sg_a, sg_b):
        wid = lax.axis_index("s") * NC + lax.axis_index("c")
        i0 = (i0a, i0b, i0c)
        wxv = (wxa, wxb, wxc)
        wyv = (wya, wyb, wyc)
        corners = ((ca0, ca1, ca2, ca3), (cb0, cb1, cb2, cb3))
        oc = (oca, ocb)
        sem_in = (sin_a, sin_b, sin_c)
        sem_g = (sg_a, sg_b)
        tbls = (table,
                table.at[pl.ds(1, TBL - 1)],
                table.at[pl.ds(R, TBL - R)],
                table.at[pl.ds(R + 1, TBL - R - 1)])

        def fire_in(g, p):
            base = p * n_pad + wid * nw_pts + g * CH
            pltpu.async_copy(idxh.at[pl.ds(base, CH)], i0[p], sem_in[p])
            pltpu.async_copy(wxh.at[pl.ds(base, CH)], wxv[p], sem_in[p])
            pltpu.async_copy(wyh.at[pl.ds(base, CH)], wyv[p], sem_in[p])

        def wait_in(p):
            pltpu.make_async_copy(idxh.at[pl.ds(0, CH)], i0[p],
                                  sem_in[p]).wait()
            pltpu.make_async_copy(wxh.at[pl.ds(0, CH)], wxv[p],
                                  sem_in[p]).wait()
            pltpu.make_async_copy(wyh.at[pl.ds(0, CH)], wyv[p],
                                  sem_in[p]).wait()

        def fire_gathers(p, par):
            for c in range(4):
                for h in range(2):
                    iv = i0[p].at[pl.ds(h * HC, HC)]
                    dv = corners[par][c].at[pl.ds(h * HC, HC), :]
                    pltpu.async_copy(tbls[c].at[iv], dv, sem_g[par])

        def wait_gathers(par):
            for c in range(4):
                pltpu.make_async_copy(table.at[pl.ds(0, CH)],
                                      corners[par][c], sem_g[par]).wait()

        def combine(p, par, poc):
            c0, c1, c2, c3 = corners[par]
            ocr = oc[poc]
            wxr = wxv[p]
            wyr = wyv[p]

            def grp(t, cin):
                s16 = pl.ds(t * L, L)
                wxg = wxr[s16]
                wyg = wyr[s16]
                ax = 1.0 - wxg
                ay = 1.0 - wyg
                a0 = ax * ay
                a1 = wxg * ay
                a2 = ax * wyg
                a3 = wxg * wyg
                for u in range(L):
                    pt = t * L + u
                    acc = (c0[pt, :] * a0[u] + c1[pt, :] * a1[u]
                           + c2[pt, :] * a2[u] + c3[pt, :] * a3[u])
                    ocr[pt, pl.ds(p * F, F)] = acc
                return cin

            lax.fori_loop(0, CH // L, grp, 0)

        def store(g, poc):
            base = wid * nw_pts + g * CH
            pltpu.sync_copy(oc[poc], outh.at[pl.ds(base, CH), :])

        fire_in(0, 0)
        fire_in(0, 1)
        fire_in(0, 2)
        wait_in(0)
        fire_gathers(0, 0)

        def body(k2, carry):
            for j in range(6):
                wait_in((1 + j) % 3)
                fire_gathers((1 + j) % 3, (1 + j) % 2)
                wait_gathers(j % 2)
                combine(j % 3, j % 2, (j // 3) % 2)
                fire_in(2 * k2 + 1 + j // 3, j % 3)
                if j % 3 == 2:
                    store(2 * k2 + j // 3, (j // 3) % 2)
            return carry

        lax.fori_loop(0, loop_iters, body, 0)

        for s in range(6 * loop_iters + 1, n_steps):
            wait_in(s % 3)
            fire_gathers(s % 3, s % 2)
            sp = s - 1
            wait_gathers(sp % 2)
            combine(sp % 3, sp % 2, (sp // 3) % 2)
            if s + 2 < n_steps:
                fire_in((s + 2) // 3, (s + 2) % 3)
            if sp % 3 == 2:
                store(sp // 3, (sp // 3) % 2)
        sp = n_steps - 1
        wait_gathers(sp % 2)
        combine(sp % 3, sp % 2, (sp // 3) % 2)
        store(sp // 3, (sp // 3) % 2)

    return sc_kernel


def kernel(x, plane_xy, plane_xz, plane_yz):
    n = x.shape[0]
    blk = NW * CH * 2
    n_pad = ((n + blk - 1) // blk) * blk
    n_pad = ((n_pad + BN - 1) // BN) * BN

    table = jnp.concatenate(
        [jnp.transpose(p[0], (1, 2, 0)).reshape(R * R, F)
         for p in (plane_xy, plane_xz, plane_yz)], axis=0)

    x_t = jnp.pad(jnp.transpose(x), ((0, 0), (0, n_pad - n)), mode="edge")

    grid = n_pad // BN
    mn, mx = pl.pallas_call(
        _minmax_body,
        grid=(grid,),
        in_specs=[pl.BlockSpec((3, BN), lambda i: (0, i))],
        out_specs=[pl.BlockSpec((3, 128), lambda i: (0, 0)),
                   pl.BlockSpec((3, 128), lambda i: (0, 0))],
        out_shape=[jax.ShapeDtypeStruct((3, 128), jnp.float32),
                   jax.ShapeDtypeStruct((3, 128), jnp.float32)],
    )(x_t)

    idx, wx, wy = pl.pallas_call(
        _idxw_body,
        grid=(grid,),
        in_specs=[pl.BlockSpec((3, BN), lambda i: (0, i)),
                  pl.BlockSpec((3, 128), lambda i: (0, 0)),
                  pl.BlockSpec((3, 128), lambda i: (0, 0))],
        out_specs=[pl.BlockSpec((3, BN), lambda i: (0, i)),
                   pl.BlockSpec((3, BN), lambda i: (0, i)),
                   pl.BlockSpec((3, BN), lambda i: (0, i))],
        out_shape=[jax.ShapeDtypeStruct((3, n_pad), jnp.int32),
                   jax.ShapeDtypeStruct((3, n_pad), jnp.float32),
                   jax.ShapeDtypeStruct((3, n_pad), jnp.float32)],
    )(x_t, mn, mx)

    out = _make_sc_kernel(n_pad)(table, idx.reshape(-1), wx.reshape(-1),
                                 wy.reshape(-1))
    return out[:n]
